# baseline (speedup 1.0000x reference)
import numpy as np
import jax
import jax.numpy as jnp
from jax.experimental import pallas as pl
from jax.experimental.pallas import tpu as pltpu

_H = 76
_W = 76
_A = 3
_F = 6
_N = 20
_HW = _H * _W
_W3 = _A * _W


def _sigmoid(x):
    return jax.nn.sigmoid(x)


def _clamped_log(p):
    return jnp.maximum(jnp.log(jnp.maximum(p, 1e-12)), -100.0)


_ATAN_C = (9.999998978e-01, -3.333195972e-01, 1.996923539e-01,
           -1.401658504e-01, 9.906096896e-02, -5.936710079e-02,
           2.416618952e-02, -4.668773308e-03)


def _atan_ratio(w, h):
    hh = jnp.maximum(h, 1e-6)
    lo = jnp.minimum(w, hh)
    hi = jnp.maximum(w, hh)
    u = lo / hi
    q = u * u
    p = jnp.float32(_ATAN_C[7])
    for c in _ATAN_C[6::-1]:
        p = p * q + jnp.float32(c)
    at = u * p
    return jnp.where(w > hh, jnp.float32(np.pi / 2) - at, at)


def _make_loss_body(img):
    return lambda boxes_ref, anchors_ref, inf_ref, out_ref: _loss_common(
        img, boxes_ref, anchors_ref, inf_ref, out_ref)


def _loss_common(img, boxes_ref, anchors_ref, inf_ref, out_ref):
    hr = img * _H
    lane = jax.lax.broadcasted_iota(
        jnp.int32, (hr, _W3), 1).astype(jnp.float32)
    row = jax.lax.broadcasted_iota(
        jnp.int32, (hr, _W3), 0).astype(jnp.float32)
    af = jnp.where(lane >= float(2 * _W), 2.0,
                   jnp.where(lane >= float(_W), 1.0, 0.0))
    gx = lane - af * float(_W)
    imf = jnp.zeros_like(row)
    for i in range(1, img):
        imf = jnp.where(row >= float(i * _H), float(i), imf)
    gy = row - imf * float(_H)
    cell3 = (imf * float(_A * _HW) + af * float(_HW)
             + gy * float(_W) + gx)
    imgmask = [imf == float(i) for i in range(img)]

    def apick(vals):
        return jnp.where(af == 2.0, vals[2],
                         jnp.where(af == 1.0, vals[1], vals[0]))

    awp = apick([anchors_ref[a, 0] for a in range(_A)])
    ahp = apick([anchors_ref[a, 1] for a in range(_A)])

    def cat(f):
        return jnp.concatenate(
            [jnp.concatenate([inf_ref[i, a * _F + f] for a in range(_A)],
                             axis=1) for i in range(img)], axis=0)

    obj_p = _sigmoid(cat(0))
    cx = 1.2 * _sigmoid(cat(1)) - 0.1
    cy = 1.2 * _sigmoid(cat(2)) - 0.1
    bx = (cx + gx) / float(_W)
    by = (cy + gy) / float(_H)
    bw = jnp.exp(cat(3)) * awp
    bh = jnp.exp(cat(4)) * ahp
    cls_p = _sigmoid(cat(5))

    px1 = bx - bw / 2
    py1 = by - bh / 2
    px2 = bx + bw / 2
    py2 = by + bh / 2
    area_p = (px2 - px1) * (py2 - py1)

    hit = jnp.zeros_like(gx, dtype=jnp.bool_)
    tb0 = jnp.zeros_like(gx)
    tb1 = jnp.zeros_like(gx)
    tb2 = jnp.zeros_like(gx)
    tb3 = jnp.zeros_like(gx)

    anc_w = [anchors_ref[a, 0] for a in range(_A)]
    anc_h = [anchors_ref[a, 1] for a in range(_A)]
    anc_area = [anc_w[a] * anc_h[a] for a in range(_A)]

    for i in range(img):
      for k in range(_N):
        b0 = boxes_ref[i, k, 0]
        b1 = boxes_ref[i, k, 1]
        b2 = boxes_ref[i, k, 2]
        b3 = boxes_ref[i, k, 3]
        gif = (b0 * float(_W)).astype(jnp.int32).astype(jnp.float32)
        gjf = (b1 * float(_H)).astype(jnp.int32).astype(jnp.float32)
        bb = b2 * b3
        best_i = jnp.minimum(b2, anc_w[0]) * jnp.minimum(b3, anc_h[0])
        best_u = bb + anc_area[0] - best_i
        best_a = jnp.float32(0.0)
        for a in range(1, _A):
            inter = jnp.minimum(b2, anc_w[a]) * jnp.minimum(b3, anc_h[a])
            union = bb + anc_area[a] - inter
            upd = inter * best_u > best_i * union
            best_a = jnp.where(upd, jnp.float32(a), best_a)
            best_i = jnp.where(upd, inter, best_i)
            best_u = jnp.where(upd, union, best_u)
        cell_k = (float(i * _A * _HW) + best_a * float(_HW)
                  + gjf * float(_W) + gif)
        mask_k = cell3 == cell_k
        tb0 = jnp.where(mask_k, b0, tb0)
        tb1 = jnp.where(mask_k, b1, tb1)
        tb2 = jnp.where(mask_k, b2, tb2)
        tb3 = jnp.where(mask_k, b3, tb3)
        tx1 = b0 - b2 / 2
        ty1 = b1 - b3 / 2
        tx2 = b0 + b2 / 2
        ty2 = b1 + b3 / 2
        area_t = (tx2 - tx1) * (ty2 - ty1)
        iw2 = jnp.minimum(tx2, px2) - jnp.maximum(tx1, px1)
        ih2 = jnp.maximum(jnp.minimum(ty2, py2) - jnp.maximum(ty1, py1), 0.0)
        inter2 = iw2 * ih2
        hit = hit | ((3.0 * inter2 > area_t + area_p) & imgmask[i])

    tobj = jnp.where(tb2 > 0.0, 1.0, 0.0)
    noobj = jnp.where(hit, 0.0, 1.0)

    lp = _clamped_log(obj_p)
    lq = _clamped_log(1.0 - obj_p)
    bce_obj = -(tobj * lp + (1.0 - tobj) * lq)
    bce_cls1 = -_clamped_log(cls_p)

    tx1 = tb0 - tb2 / 2
    ty1 = tb1 - tb3 / 2
    tx2 = tb0 + tb2 / 2
    ty2 = tb1 + tb3 / 2
    iw = jnp.maximum(jnp.minimum(px2, tx2) - jnp.maximum(px1, tx1), 0.0)
    ih = jnp.maximum(jnp.minimum(py2, ty2) - jnp.maximum(py1, ty1), 0.0)
    inter = iw * ih
    union = bw * bh + tb2 * tb3 - inter
    iou = inter / jnp.maximum(union, 1e-6)
    d = jnp.abs(bx - tb0) + jnp.abs(by - tb1)
    cenc = (jnp.maximum(px2, tx2) - jnp.minimum(px1, tx1)
            + jnp.maximum(py2, ty2) - jnp.minimum(py1, ty1))
    dis = d / jnp.maximum(cenc, 1e-6)
    a1 = _atan_ratio(bw, bh)
    a2 = _atan_ratio(tb2, tb3)
    v = 4.0 / (3.1415926 ** 2) * jnp.abs(a1 - a2)
    aa = v / jnp.maximum(1.0 - iou + v, 1e-6)
    ciou = 1.0 - iou + dis + aa * v

    out_ref[0, 0, 0] = jnp.sum(tobj)
    out_ref[0, 0, 1] = jnp.sum(tobj * ciou)
    out_ref[0, 0, 2] = jnp.sum(tobj * bce_obj)
    out_ref[0, 0, 3] = jnp.sum(noobj * bce_obj)
    out_ref[0, 0, 4] = jnp.sum(noobj)
    out_ref[0, 0, 5] = jnp.sum(tobj * bce_cls1)
    out_ref[0, 0, 6] = 0.0
    out_ref[0, 0, 7] = 0.0


def _partials(yolo_head, boxes, anchors, interpret=False, img=2):
    bs = yolo_head.shape[0]
    if bs % img != 0:
        img = 1
    return pl.pallas_call(
        _make_loss_body(img),
        grid=(bs // img,),
        in_specs=[
            pl.BlockSpec((img, _N, 4), lambda b: (b, 0, 0),
                         memory_space=pltpu.SMEM),
            pl.BlockSpec((_A, 2), lambda b: (0, 0),
                         memory_space=pltpu.SMEM),
            pl.BlockSpec((img, _A * _F, _H, _W), lambda b: (b, 0, 0, 0)),
        ],
        out_specs=pl.BlockSpec((1, 1, 8), lambda b: (b, 0, 0),
                               memory_space=pltpu.SMEM),
        out_shape=jax.ShapeDtypeStruct((bs // img, 1, 8), jnp.float32),
        interpret=interpret,
    )(boxes, anchors, yolo_head)


def kernel(yolo_head, boxes, labels, anchors):
    del labels
    p = _partials(yolo_head, boxes, anchors)
    t = jnp.sum(p, axis=(0, 1))
    K = t[0]
    box_loss = 0.05 * t[1] / K
    cls_loss = t[5] / K
    grid_loss = (1.5 * t[2] + 0.5 * t[3]) / (K + t[4]) + 0.5 * cls_loss
    return box_loss, grid_loss


# speedup vs baseline: 1.5840x; 1.5840x over previous
import numpy as np
import jax
import jax.numpy as jnp
from jax.experimental import pallas as pl
from jax.experimental.pallas import tpu as pltpu

_H = 76
_W = 76
_A = 3
_F = 6
_N = 20
_HW = _H * _W
_W3 = _A * _W


def _sigmoid(x):
    return jax.nn.sigmoid(x)


def _clamped_log(p):
    return jnp.maximum(jnp.log(jnp.maximum(p, 1e-12)), -100.0)


_ATAN_C = (9.999998978e-01, -3.333195972e-01, 1.996923539e-01,
           -1.401658504e-01, 9.906096896e-02, -5.936710079e-02,
           2.416618952e-02, -4.668773308e-03)


def _atan_ratio(w, h):
    hh = jnp.maximum(h, 1e-6)
    lo = jnp.minimum(w, hh)
    hi = jnp.maximum(w, hh)
    u = lo / hi
    q = u * u
    p = jnp.float32(_ATAN_C[7])
    for c in _ATAN_C[6::-1]:
        p = p * q + jnp.float32(c)
    at = u * p
    return jnp.where(w > hh, jnp.float32(np.pi / 2) - at, at)


def _make_loss_body(img):
    return lambda boxes_ref, anchors_ref, inf_ref, out_ref: _loss_common(
        img, boxes_ref, anchors_ref, inf_ref, out_ref)


def _loss_common(img, boxes_ref, anchors_ref, inf_ref, out_ref):
    hr = img * _H
    lane = jax.lax.broadcasted_iota(
        jnp.int32, (hr, _W3), 1).astype(jnp.float32)
    row = jax.lax.broadcasted_iota(
        jnp.int32, (hr, _W3), 0).astype(jnp.float32)
    af = jnp.where(lane >= float(2 * _W), 2.0,
                   jnp.where(lane >= float(_W), 1.0, 0.0))
    gx = lane - af * float(_W)
    if img == 1:
        gy = row
        cell3 = af * float(_HW) + gy * float(_W) + gx
        imgmask = None
    else:
        imf = jnp.zeros_like(row)
        for i in range(1, img):
            imf = jnp.where(row >= float(i * _H), float(i), imf)
        gy = row - imf * float(_H)
        cell3 = (imf * float(_A * _HW) + af * float(_HW)
                 + gy * float(_W) + gx)
        imgmask = [imf == float(i) for i in range(img)]

    def apick(vals):
        return jnp.where(af == 2.0, vals[2],
                         jnp.where(af == 1.0, vals[1], vals[0]))

    awp = apick([anchors_ref[a, 0] for a in range(_A)])
    ahp = apick([anchors_ref[a, 1] for a in range(_A)])

    def cat(f):
        return jnp.concatenate(
            [jnp.concatenate([inf_ref[i, a * _F + f] for a in range(_A)],
                             axis=1) for i in range(img)], axis=0)

    obj_p = _sigmoid(cat(0))
    cx = 1.2 * _sigmoid(cat(1)) - 0.1
    cy = 1.2 * _sigmoid(cat(2)) - 0.1
    bx = (cx + gx) / float(_W)
    by = (cy + gy) / float(_H)
    bw = jnp.exp(cat(3)) * awp
    bh = jnp.exp(cat(4)) * ahp
    cls_p = _sigmoid(cat(5))

    px1 = bx - bw / 2
    py1 = by - bh / 2
    px2 = bx + bw / 2
    py2 = by + bh / 2
    area_p = (px2 - px1) * (py2 - py1)

    hit = jnp.zeros_like(gx, dtype=jnp.bool_)
    tb0 = jnp.zeros_like(gx)
    tb1 = jnp.zeros_like(gx)
    tb2 = jnp.zeros_like(gx)
    tb3 = jnp.zeros_like(gx)

    anc_w = [anchors_ref[a, 0] for a in range(_A)]
    anc_h = [anchors_ref[a, 1] for a in range(_A)]
    anc_area = [anc_w[a] * anc_h[a] for a in range(_A)]

    for i in range(img):
      for k in range(_N):
        b0 = boxes_ref[i, k, 0]
        b1 = boxes_ref[i, k, 1]
        b2 = boxes_ref[i, k, 2]
        b3 = boxes_ref[i, k, 3]
        gif = (b0 * float(_W)).astype(jnp.int32).astype(jnp.float32)
        gjf = (b1 * float(_H)).astype(jnp.int32).astype(jnp.float32)
        bb = b2 * b3
        best_i = jnp.minimum(b2, anc_w[0]) * jnp.minimum(b3, anc_h[0])
        best_u = bb + anc_area[0] - best_i
        best_a = jnp.float32(0.0)
        for a in range(1, _A):
            inter = jnp.minimum(b2, anc_w[a]) * jnp.minimum(b3, anc_h[a])
            union = bb + anc_area[a] - inter
            upd = inter * best_u > best_i * union
            best_a = jnp.where(upd, jnp.float32(a), best_a)
            best_i = jnp.where(upd, inter, best_i)
            best_u = jnp.where(upd, union, best_u)
        cell_k = (float(i * _A * _HW) + best_a * float(_HW)
                  + gjf * float(_W) + gif)
        mask_k = cell3 == cell_k
        tb0 = jnp.where(mask_k, b0, tb0)
        tb1 = jnp.where(mask_k, b1, tb1)
        tb2 = jnp.where(mask_k, b2, tb2)
        tb3 = jnp.where(mask_k, b3, tb3)
        tx1 = b0 - b2 / 2
        ty1 = b1 - b3 / 2
        tx2 = b0 + b2 / 2
        ty2 = b1 + b3 / 2
        area_t = (tx2 - tx1) * (ty2 - ty1)
        iw2 = jnp.minimum(tx2, px2) - jnp.maximum(tx1, px1)
        ih2 = jnp.maximum(jnp.minimum(ty2, py2) - jnp.maximum(ty1, py1), 0.0)
        inter2 = iw2 * ih2
        hc = 3.0 * inter2 > area_t + area_p
        if img > 1:
            hc = hc & imgmask[i]
        hit = hit | hc

    tobj = jnp.where(tb2 > 0.0, 1.0, 0.0)
    noobj = jnp.where(hit, 0.0, 1.0)

    lp = _clamped_log(obj_p)
    lq = _clamped_log(1.0 - obj_p)
    bce_obj = -(tobj * lp + (1.0 - tobj) * lq)
    bce_cls1 = -_clamped_log(cls_p)

    tx1 = tb0 - tb2 / 2
    ty1 = tb1 - tb3 / 2
    tx2 = tb0 + tb2 / 2
    ty2 = tb1 + tb3 / 2
    iw = jnp.maximum(jnp.minimum(px2, tx2) - jnp.maximum(px1, tx1), 0.0)
    ih = jnp.maximum(jnp.minimum(py2, ty2) - jnp.maximum(py1, ty1), 0.0)
    inter = iw * ih
    union = bw * bh + tb2 * tb3 - inter
    iou = inter / jnp.maximum(union, 1e-6)
    d = jnp.abs(bx - tb0) + jnp.abs(by - tb1)
    cenc = (jnp.maximum(px2, tx2) - jnp.minimum(px1, tx1)
            + jnp.maximum(py2, ty2) - jnp.minimum(py1, ty1))
    dis = d / jnp.maximum(cenc, 1e-6)
    a1 = _atan_ratio(bw, bh)
    a2 = _atan_ratio(tb2, tb3)
    v = 4.0 / (3.1415926 ** 2) * jnp.abs(a1 - a2)
    aa = v / jnp.maximum(1.0 - iou + v, 1e-6)
    ciou = 1.0 - iou + dis + aa * v

    out_ref[0, 0, 0] = jnp.sum(tobj)
    out_ref[0, 0, 1] = jnp.sum(tobj * ciou)
    out_ref[0, 0, 2] = jnp.sum(tobj * bce_obj)
    out_ref[0, 0, 3] = jnp.sum(noobj * bce_obj)
    out_ref[0, 0, 4] = jnp.sum(noobj)
    out_ref[0, 0, 5] = jnp.sum(tobj * bce_cls1)
    out_ref[0, 0, 6] = 0.0
    out_ref[0, 0, 7] = 0.0


def _partials(yolo_head, boxes, anchors, interpret=False, img=1):
    bs = yolo_head.shape[0]
    if bs % img != 0:
        img = 1
    return pl.pallas_call(
        _make_loss_body(img),
        grid=(bs // img,),
        in_specs=[
            pl.BlockSpec((img, _N, 4), lambda b: (b, 0, 0),
                         memory_space=pltpu.SMEM),
            pl.BlockSpec((_A, 2), lambda b: (0, 0),
                         memory_space=pltpu.SMEM),
            pl.BlockSpec((img, _A * _F, _H, _W), lambda b: (b, 0, 0, 0)),
        ],
        out_specs=pl.BlockSpec((1, 1, 8), lambda b: (b, 0, 0),
                               memory_space=pltpu.SMEM),
        out_shape=jax.ShapeDtypeStruct((bs // img, 1, 8), jnp.float32),
        interpret=interpret,
    )(boxes, anchors, yolo_head)


def kernel(yolo_head, boxes, labels, anchors):
    del labels
    p = _partials(yolo_head, boxes, anchors)
    t = jnp.sum(p, axis=(0, 1))
    K = t[0]
    box_loss = 0.05 * t[1] / K
    cls_loss = t[5] / K
    grid_loss = (1.5 * t[2] + 0.5 * t[3]) / (K + t[4]) + 0.5 * cls_loss
    return box_loss, grid_loss
